# baseline (device time: 103515 ns/iter reference)
import jax
import jax.numpy as jnp
from jax import lax
from jax.experimental import pallas as pl
from jax.experimental.pallas import tpu as pltpu

N_DEV = 4
T = 4096
D = 1024
DR, DC = 32, 128
PAD = 1088
SUB = PAD // 4


def _dest_gather_body(d_ref, dall_ref, send_sems, recv_sems, local_sem):
    my_x = lax.axis_index("x")
    my_y = lax.axis_index("y")
    my_z = lax.axis_index("z")

    barrier = pltpu.get_barrier_semaphore()
    for o in range(1, N_DEV):
        nbr = lax.rem(my_z + o, N_DEV)
        pl.semaphore_signal(
            barrier, inc=1,
            device_id=(my_x, my_y, nbr),
            device_id_type=pl.DeviceIdType.MESH,
        )
    pl.semaphore_wait(barrier, N_DEV - 1)

    cp = pltpu.make_async_copy(d_ref, dall_ref.at[my_z], local_sem)
    cp.start()

    sends = []
    for o in range(1, N_DEV):
        r = lax.rem(my_z + o, N_DEV)
        rdma = pltpu.make_async_remote_copy(
            src_ref=d_ref,
            dst_ref=dall_ref.at[my_z],
            send_sem=send_sems.at[r],
            recv_sem=recv_sems.at[my_z],
            device_id=(my_x, my_y, r),
            device_id_type=pl.DeviceIdType.MESH,
        )
        rdma.start()
        sends.append(rdma)

    cp.wait()
    for s in range(N_DEV):
        @pl.when(my_z != s)
        def _():
            rx = pltpu.make_async_remote_copy(
                src_ref=d_ref,
                dst_ref=dall_ref.at[s],
                send_sem=send_sems.at[s],
                recv_sem=recv_sems.at[s],
                device_id=(my_x, my_y, my_z),
                device_id_type=pl.DeviceIdType.MESH,
            )
            rx.wait_recv()

    for rdma in sends:
        rdma.wait_send()


def _a2av_body(
    x_ref, ls_ref, starts_ref, ends_ref, offs_ref,
    out_ref,
    xsorted, recv_buf, send_sems, recv_sems, local_sem,
):
    my_x = lax.axis_index("x")
    my_y = lax.axis_index("y")
    my_z = lax.axis_index("z")

    barrier = pltpu.get_barrier_semaphore()
    for o in range(1, N_DEV):
        nbr = lax.rem(my_z + o, N_DEV)
        pl.semaphore_signal(
            barrier, inc=1,
            device_id=(my_x, my_y, nbr),
            device_id_type=pl.DeviceIdType.MESH,
        )
    pl.semaphore_signal(
        barrier, inc=1,
        device_id=(1 - my_x, my_y, my_z),
        device_id_type=pl.DeviceIdType.MESH,
    )
    pl.semaphore_wait(barrier, N_DEV)

    def gather(k, c):
        idx = ls_ref[k]
        xsorted[pl.ds(k, 1)] = x_ref[pl.ds(idx, 1)]
        return c

    sends = []
    for o in (2, 1, 3):
        r = lax.rem(my_z + o, N_DEV)
        start = starts_ref[r]
        end = ends_ref[r]
        for k in range(N_DEV):
            if o == 2:
                send_this = k // 2 == my_x
            else:
                send_this = None
            lo = jnp.minimum(start + SUB * k, end)
            hi = jnp.minimum(start + SUB * (k + 1), end)
            rdma = pltpu.make_async_remote_copy(
                src_ref=xsorted.at[pl.ds(start + SUB * k, SUB)],
                dst_ref=recv_buf.at[my_z, pl.ds(SUB * k, SUB)],
                send_sem=send_sems.at[r, k],
                recv_sem=recv_sems.at[my_z, k],
                device_id=(my_x, my_y, r),
                device_id_type=pl.DeviceIdType.MESH,
            )
            if send_this is None:
                lax.fori_loop(lo, hi, gather, 0)
                rdma.start()
                sends.append(rdma)
            else:
                @pl.when(send_this)
                def _():
                    lax.fori_loop(lo, hi, gather, 0)
                    rdma.start()
                sends.append((rdma, send_this))

    lax.fori_loop(starts_ref[my_z], ends_ref[my_z], gather, 0)

    s2 = lax.rem(my_z + 2, N_DEV)
    for k in range(N_DEV):
        mine_half = k // 2 == my_x

        @pl.when(mine_half)
        def _():
            rx = pltpu.make_async_remote_copy(
                src_ref=recv_buf.at[s2, pl.ds(SUB * k, SUB)],
                dst_ref=recv_buf.at[s2, pl.ds(SUB * k, SUB)],
                send_sem=send_sems.at[my_z, k],
                recv_sem=recv_sems.at[s2, k],
                device_id=(my_x, my_y, my_z),
                device_id_type=pl.DeviceIdType.MESH,
            )
            rx.wait_recv()
            fwd = pltpu.make_async_remote_copy(
                src_ref=recv_buf.at[s2, pl.ds(SUB * k, SUB)],
                dst_ref=recv_buf.at[s2, pl.ds(SUB * k, SUB)],
                send_sem=send_sems.at[my_z, k],
                recv_sem=recv_sems.at[s2, k],
                device_id=(1 - my_x, my_y, my_z),
                device_id_type=pl.DeviceIdType.MESH,
            )
            fwd.start()
        sends.append((
            pltpu.make_async_remote_copy(
                src_ref=recv_buf.at[s2, pl.ds(SUB * k, SUB)],
                dst_ref=recv_buf.at[s2, pl.ds(SUB * k, SUB)],
                send_sem=send_sems.at[my_z, k],
                recv_sem=recv_sems.at[s2, k],
                device_id=(1 - my_x, my_y, my_z),
                device_id_type=pl.DeviceIdType.MESH,
            ),
            mine_half,
        ))

    for s in range(N_DEV):
        @pl.when(my_z == s)
        def _own():
            cp = pltpu.make_async_copy(
                xsorted.at[pl.ds(starts_ref[s], PAD)],
                out_ref.at[pl.ds(offs_ref[s], PAD)],
                local_sem,
            )
            cp.start()
            cp.wait()

        @pl.when(my_z != s)
        def _recv():
            for k in range(N_DEV):
                already = jnp.logical_and(s2 == s, k // 2 == my_x)

                @pl.when(jnp.logical_not(already))
                def _():
                    rx = pltpu.make_async_remote_copy(
                        src_ref=recv_buf.at[s, pl.ds(SUB * k, SUB)],
                        dst_ref=recv_buf.at[s, pl.ds(SUB * k, SUB)],
                        send_sem=send_sems.at[s, k],
                        recv_sem=recv_sems.at[s, k],
                        device_id=(my_x, my_y, my_z),
                        device_id_type=pl.DeviceIdType.MESH,
                    )
                    rx.wait_recv()
            cp = pltpu.make_async_copy(
                recv_buf.at[s],
                out_ref.at[pl.ds(offs_ref[s], PAD)],
                local_sem,
            )
            cp.start()
            cp.wait()

    for item in sends:
        if isinstance(item, tuple):
            rdma, cond = item

            @pl.when(cond)
            def _():
                rdma.wait_send()
        else:
            item.wait_send()


def kernel(x, dest):
    d2 = dest.reshape(DR, DC)
    dall = pl.pallas_call(
        _dest_gather_body,
        out_shape=jax.ShapeDtypeStruct((N_DEV, DR, DC), jnp.int32),
        in_specs=[pl.BlockSpec(memory_space=pltpu.VMEM)],
        out_specs=pl.BlockSpec(memory_space=pltpu.VMEM),
        scratch_shapes=[
            pltpu.SemaphoreType.DMA((N_DEV,)),
            pltpu.SemaphoreType.DMA((N_DEV,)),
            pltpu.SemaphoreType.DMA,
        ],
        compiler_params=pltpu.CompilerParams(collective_id=0),
    )(d2)

    my_z = lax.axis_index("z")
    dest_all = dall.reshape(N_DEV, T)
    counts = (dest_all[:, :, None] == jnp.arange(N_DEV)[None, None, :]).sum(
        axis=1, dtype=jnp.int32
    )
    my_counts = lax.dynamic_slice(counts, (my_z, 0), (1, N_DEV)).reshape(N_DEV)
    col_counts = lax.dynamic_slice(counts, (0, my_z), (N_DEV, 1)).reshape(N_DEV)
    zero = jnp.zeros((1,), jnp.int32)
    starts = jnp.concatenate([zero, jnp.cumsum(my_counts)[:-1]]).astype(jnp.int32)
    ends = (starts + my_counts).astype(jnp.int32)
    offs = jnp.concatenate([zero, jnp.cumsum(col_counts)[:-1]]).astype(jnp.int32)
    ls = jnp.argsort(dest, stable=True).astype(jnp.int32)

    x3 = x.astype(jnp.bfloat16).reshape(T, 8, 128)
    out3 = pl.pallas_call(
        _a2av_body,
        out_shape=jax.ShapeDtypeStruct((T + PAD, 8, 128), jnp.bfloat16),
        in_specs=[
            pl.BlockSpec(memory_space=pltpu.VMEM),
            pl.BlockSpec(memory_space=pltpu.SMEM),
            pl.BlockSpec(memory_space=pltpu.SMEM),
            pl.BlockSpec(memory_space=pltpu.SMEM),
            pl.BlockSpec(memory_space=pltpu.SMEM),
        ],
        out_specs=pl.BlockSpec(memory_space=pltpu.VMEM),
        scratch_shapes=[
            pltpu.VMEM((T + PAD, 8, 128), jnp.bfloat16),
            pltpu.VMEM((N_DEV, PAD, 8, 128), jnp.bfloat16),
            pltpu.SemaphoreType.DMA((N_DEV, N_DEV)),
            pltpu.SemaphoreType.DMA((N_DEV, N_DEV)),
            pltpu.SemaphoreType.DMA,
        ],
        compiler_params=pltpu.CompilerParams(collective_id=1),
    )(x3, ls, starts, ends, offs)

    return out3[:T].reshape(T, D)


# device time: 92532 ns/iter; 1.1187x vs baseline; 1.1187x over previous
import jax
import jax.numpy as jnp
from jax import lax
from jax.experimental import pallas as pl
from jax.experimental.pallas import tpu as pltpu

N_DEV = 4
T = 4096
D = 1024
PAD = 1088
SUB = PAD // 4


def _a2av_body(
    x_ref, ls_ref, starts_ref, ends_ref, cnt_ref,
    out_ref,
    xsorted, recv_buf, cnt_all, cnt_smem,
    send_sems, recv_sems, cnt_send, cnt_recv, local_sem, cnt_local,
):
    my_x = lax.axis_index("x")
    my_y = lax.axis_index("y")
    my_z = lax.axis_index("z")

    barrier = pltpu.get_barrier_semaphore()
    for o in range(1, N_DEV):
        nbr = lax.rem(my_z + o, N_DEV)
        pl.semaphore_signal(
            barrier, inc=1,
            device_id=(my_x, my_y, nbr),
            device_id_type=pl.DeviceIdType.MESH,
        )
    pl.semaphore_signal(
        barrier, inc=1,
        device_id=(1 - my_x, my_y, my_z),
        device_id_type=pl.DeviceIdType.MESH,
    )
    pl.semaphore_wait(barrier, N_DEV)

    cnt_sends = []
    for o in range(1, N_DEV):
        r = lax.rem(my_z + o, N_DEV)
        rdma = pltpu.make_async_remote_copy(
            src_ref=cnt_ref,
            dst_ref=cnt_all.at[pl.ds(my_z, 1)],
            send_sem=cnt_send.at[r],
            recv_sem=cnt_recv.at[my_z],
            device_id=(my_x, my_y, r),
            device_id_type=pl.DeviceIdType.MESH,
        )
        rdma.start()
        cnt_sends.append(rdma)

    def gather(k, c):
        idx = ls_ref[k]
        xsorted[pl.ds(k, 1)] = x_ref[pl.ds(idx, 1)]
        return c

    sends = []
    for o in (1, 2, 3):
        r = lax.rem(my_z + o, N_DEV)
        start = starts_ref[r]
        end = ends_ref[r]
        for k in range(N_DEV):
            if o == 2:
                send_this = k // 2 == my_x
            else:
                send_this = None
            lo = jnp.minimum(start + SUB * k, end)
            hi = jnp.minimum(start + SUB * (k + 1), end)
            rdma = pltpu.make_async_remote_copy(
                src_ref=xsorted.at[pl.ds(start + SUB * k, SUB)],
                dst_ref=recv_buf.at[my_z, pl.ds(SUB * k, SUB)],
                send_sem=send_sems.at[r, k],
                recv_sem=recv_sems.at[my_z, k],
                device_id=(my_x, my_y, r),
                device_id_type=pl.DeviceIdType.MESH,
            )
            if send_this is None:
                lax.fori_loop(lo, hi, gather, 0)
                rdma.start()
                sends.append(rdma)
            else:
                @pl.when(send_this)
                def _():
                    lax.fori_loop(lo, hi, gather, 0)
                    rdma.start()
                sends.append((rdma, send_this))

    lax.fori_loop(starts_ref[my_z], ends_ref[my_z], gather, 0)

    s2 = lax.rem(my_z + 2, N_DEV)
    for k in range(N_DEV):
        mine_half = k // 2 == my_x

        @pl.when(mine_half)
        def _():
            rx = pltpu.make_async_remote_copy(
                src_ref=recv_buf.at[s2, pl.ds(SUB * k, SUB)],
                dst_ref=recv_buf.at[s2, pl.ds(SUB * k, SUB)],
                send_sem=send_sems.at[my_z, k],
                recv_sem=recv_sems.at[s2, k],
                device_id=(my_x, my_y, my_z),
                device_id_type=pl.DeviceIdType.MESH,
            )
            rx.wait_recv()
            fwd = pltpu.make_async_remote_copy(
                src_ref=recv_buf.at[s2, pl.ds(SUB * k, SUB)],
                dst_ref=recv_buf.at[s2, pl.ds(SUB * k, SUB)],
                send_sem=send_sems.at[my_z, k],
                recv_sem=recv_sems.at[s2, k],
                device_id=(1 - my_x, my_y, my_z),
                device_id_type=pl.DeviceIdType.MESH,
            )
            fwd.start()
        sends.append((
            pltpu.make_async_remote_copy(
                src_ref=recv_buf.at[s2, pl.ds(SUB * k, SUB)],
                dst_ref=recv_buf.at[s2, pl.ds(SUB * k, SUB)],
                send_sem=send_sems.at[my_z, k],
                recv_sem=recv_sems.at[s2, k],
                device_id=(1 - my_x, my_y, my_z),
                device_id_type=pl.DeviceIdType.MESH,
            ),
            mine_half,
        ))

    for s in range(N_DEV):
        @pl.when(my_z != s)
        def _():
            rx = pltpu.make_async_remote_copy(
                src_ref=cnt_ref,
                dst_ref=cnt_all.at[pl.ds(s, 1)],
                send_sem=cnt_send.at[s],
                recv_sem=cnt_recv.at[s],
                device_id=(my_x, my_y, my_z),
                device_id_type=pl.DeviceIdType.MESH,
            )
            rx.wait_recv()
    cp = pltpu.make_async_copy(cnt_all, cnt_smem, cnt_local)
    cp.start()
    cp.wait()

    off = jnp.int32(0)
    for s in range(N_DEV):
        @pl.when(my_z == s)
        def _own():
            cp = pltpu.make_async_copy(
                xsorted.at[pl.ds(starts_ref[s], PAD)],
                out_ref.at[pl.ds(off, PAD)],
                local_sem,
            )
            cp.start()
            cp.wait()

        @pl.when(my_z != s)
        def _recv():
            for k in range(N_DEV):
                already = jnp.logical_and(s2 == s, k // 2 == my_x)

                @pl.when(jnp.logical_not(already))
                def _():
                    rx = pltpu.make_async_remote_copy(
                        src_ref=recv_buf.at[s, pl.ds(SUB * k, SUB)],
                        dst_ref=recv_buf.at[s, pl.ds(SUB * k, SUB)],
                        send_sem=send_sems.at[s, k],
                        recv_sem=recv_sems.at[s, k],
                        device_id=(my_x, my_y, my_z),
                        device_id_type=pl.DeviceIdType.MESH,
                    )
                    rx.wait_recv()
            cp = pltpu.make_async_copy(
                recv_buf.at[s],
                out_ref.at[pl.ds(off, PAD)],
                local_sem,
            )
            cp.start()
            cp.wait()

        c_s = jnp.where(
            my_z == s,
            ends_ref[s] - starts_ref[s],
            cnt_smem[s, my_z],
        )
        off = off + c_s

    for item in sends:
        if isinstance(item, tuple):
            rdma, cond = item

            @pl.when(cond)
            def _():
                rdma.wait_send()
        else:
            item.wait_send()
    for rdma in cnt_sends:
        rdma.wait_send()


def kernel(x, dest):
    my_counts = (dest[:, None] == jnp.arange(N_DEV)[None, :]).sum(
        axis=0, dtype=jnp.int32
    )
    zero = jnp.zeros((1,), jnp.int32)
    starts = jnp.concatenate([zero, jnp.cumsum(my_counts)[:-1]]).astype(jnp.int32)
    ends = (starts + my_counts).astype(jnp.int32)
    ls = jnp.argsort(dest, stable=True).astype(jnp.int32)
    cnt_v = my_counts.reshape(1, N_DEV)

    x3 = x.astype(jnp.bfloat16).reshape(T, 8, 128)
    out3 = pl.pallas_call(
        _a2av_body,
        out_shape=jax.ShapeDtypeStruct((T + PAD, 8, 128), jnp.bfloat16),
        in_specs=[
            pl.BlockSpec(memory_space=pltpu.VMEM),
            pl.BlockSpec(memory_space=pltpu.SMEM),
            pl.BlockSpec(memory_space=pltpu.SMEM),
            pl.BlockSpec(memory_space=pltpu.SMEM),
            pl.BlockSpec(memory_space=pltpu.VMEM),
        ],
        out_specs=pl.BlockSpec(memory_space=pltpu.VMEM),
        scratch_shapes=[
            pltpu.VMEM((T + PAD, 8, 128), jnp.bfloat16),
            pltpu.VMEM((N_DEV, PAD, 8, 128), jnp.bfloat16),
            pltpu.VMEM((N_DEV, N_DEV), jnp.int32),
            pltpu.SMEM((N_DEV, N_DEV), jnp.int32),
            pltpu.SemaphoreType.DMA((N_DEV, N_DEV)),
            pltpu.SemaphoreType.DMA((N_DEV, N_DEV)),
            pltpu.SemaphoreType.DMA((N_DEV,)),
            pltpu.SemaphoreType.DMA((N_DEV,)),
            pltpu.SemaphoreType.DMA,
            pltpu.SemaphoreType.DMA,
        ],
        compiler_params=pltpu.CompilerParams(collective_id=0),
    )(x3, ls, starts, ends, cnt_v)

    return out3[:T].reshape(T, D)


# device time: 80738 ns/iter; 1.2821x vs baseline; 1.1461x over previous
import jax
import jax.numpy as jnp
from jax import lax
from jax.experimental import pallas as pl
from jax.experimental.pallas import tpu as pltpu

N_DEV = 4
T = 4096
D = 1024
PAD = 1088
SUB = PAD // 4


def _a2av_body(
    x_ref, ls_ref, starts_ref, ends_ref, cnt_ref,
    out_ref,
    xsorted, recv_buf, cnt_all, cnt_smem,
    send_sems, recv_sems, x_fwd, y_fwd, cnt_send, cnt_recv,
    local_sem, cnt_local,
):
    my_x = lax.axis_index("x")
    my_y = lax.axis_index("y")
    my_z = lax.axis_index("z")

    barrier = pltpu.get_barrier_semaphore()
    for o in range(1, N_DEV):
        nbr = lax.rem(my_z + o, N_DEV)
        pl.semaphore_signal(
            barrier, inc=1,
            device_id=(my_x, my_y, nbr),
            device_id_type=pl.DeviceIdType.MESH,
        )
    pl.semaphore_signal(
        barrier, inc=1,
        device_id=(1 - my_x, my_y, my_z),
        device_id_type=pl.DeviceIdType.MESH,
    )
    pl.semaphore_signal(
        barrier, inc=1,
        device_id=(my_x, 1 - my_y, my_z),
        device_id_type=pl.DeviceIdType.MESH,
    )
    pl.semaphore_wait(barrier, N_DEV + 1)

    cnt_sends = []
    for o in range(1, N_DEV):
        r = lax.rem(my_z + o, N_DEV)
        rdma = pltpu.make_async_remote_copy(
            src_ref=cnt_ref,
            dst_ref=cnt_all.at[pl.ds(my_z, 1)],
            send_sem=cnt_send.at[r],
            recv_sem=cnt_recv.at[my_z],
            device_id=(my_x, my_y, r),
            device_id_type=pl.DeviceIdType.MESH,
        )
        rdma.start()
        cnt_sends.append(rdma)

    def gather(k, c):
        idx = ls_ref[k]
        xsorted[pl.ds(k, 1)] = x_ref[pl.ds(idx, 1)]
        return c

    sends = []
    for o in (1, 2, 3):
        r = lax.rem(my_z + o, N_DEV)
        start = starts_ref[r]
        end = ends_ref[r]
        for k in range(N_DEV):
            if o == 1:
                send_this = k // 2 == my_y
            elif o == 2:
                send_this = k // 2 == my_x
            else:
                send_this = None
            lo = jnp.minimum(start + SUB * k, end)
            hi = jnp.minimum(start + SUB * (k + 1), end)
            rdma = pltpu.make_async_remote_copy(
                src_ref=xsorted.at[pl.ds(start + SUB * k, SUB)],
                dst_ref=recv_buf.at[my_z, pl.ds(SUB * k, SUB)],
                send_sem=send_sems.at[r, k],
                recv_sem=recv_sems.at[my_z, k],
                device_id=(my_x, my_y, r),
                device_id_type=pl.DeviceIdType.MESH,
            )
            if send_this is None:
                lax.fori_loop(lo, hi, gather, 0)
                rdma.start()
                sends.append(rdma)
            else:
                @pl.when(send_this)
                def _():
                    lax.fori_loop(lo, hi, gather, 0)
                    rdma.start()
                sends.append((rdma, send_this))

    lax.fori_loop(starts_ref[my_z], ends_ref[my_z], gather, 0)

    s1 = lax.rem(my_z + 3, N_DEV)
    s2 = lax.rem(my_z + 2, N_DEV)
    relays = (
        (s1, my_y, (my_x, 1 - my_y, my_z), y_fwd),
        (s2, my_x, (1 - my_x, my_y, my_z), x_fwd),
    )
    for s_blk, half_sel, partner, fwd_sems in relays:
        for k in range(N_DEV):
            mine_half = k // 2 == half_sel

            @pl.when(mine_half)
            def _():
                rx = pltpu.make_async_remote_copy(
                    src_ref=recv_buf.at[s_blk, pl.ds(SUB * k, SUB)],
                    dst_ref=recv_buf.at[s_blk, pl.ds(SUB * k, SUB)],
                    send_sem=fwd_sems.at[k],
                    recv_sem=recv_sems.at[s_blk, k],
                    device_id=(my_x, my_y, my_z),
                    device_id_type=pl.DeviceIdType.MESH,
                )
                rx.wait_recv()
                fwd = pltpu.make_async_remote_copy(
                    src_ref=recv_buf.at[s_blk, pl.ds(SUB * k, SUB)],
                    dst_ref=recv_buf.at[s_blk, pl.ds(SUB * k, SUB)],
                    send_sem=fwd_sems.at[k],
                    recv_sem=recv_sems.at[s_blk, k],
                    device_id=partner,
                    device_id_type=pl.DeviceIdType.MESH,
                )
                fwd.start()
            sends.append((
                pltpu.make_async_remote_copy(
                    src_ref=recv_buf.at[s_blk, pl.ds(SUB * k, SUB)],
                    dst_ref=recv_buf.at[s_blk, pl.ds(SUB * k, SUB)],
                    send_sem=fwd_sems.at[k],
                    recv_sem=recv_sems.at[s_blk, k],
                    device_id=partner,
                    device_id_type=pl.DeviceIdType.MESH,
                ),
                mine_half,
            ))

    for s in range(N_DEV):
        @pl.when(my_z != s)
        def _():
            rx = pltpu.make_async_remote_copy(
                src_ref=cnt_ref,
                dst_ref=cnt_all.at[pl.ds(s, 1)],
                send_sem=cnt_send.at[s],
                recv_sem=cnt_recv.at[s],
                device_id=(my_x, my_y, my_z),
                device_id_type=pl.DeviceIdType.MESH,
            )
            rx.wait_recv()
    cp = pltpu.make_async_copy(cnt_all, cnt_smem, cnt_local)
    cp.start()
    cp.wait()

    off = jnp.int32(0)
    for s in range(N_DEV):
        @pl.when(my_z == s)
        def _own():
            cp = pltpu.make_async_copy(
                xsorted.at[pl.ds(starts_ref[s], PAD)],
                out_ref.at[pl.ds(off, PAD)],
                local_sem,
            )
            cp.start()
            cp.wait()

        @pl.when(my_z != s)
        def _recv():
            for k in range(N_DEV):
                already = jnp.logical_or(
                    jnp.logical_and(s1 == s, k // 2 == my_y),
                    jnp.logical_and(s2 == s, k // 2 == my_x),
                )

                @pl.when(jnp.logical_not(already))
                def _():
                    rx = pltpu.make_async_remote_copy(
                        src_ref=recv_buf.at[s, pl.ds(SUB * k, SUB)],
                        dst_ref=recv_buf.at[s, pl.ds(SUB * k, SUB)],
                        send_sem=send_sems.at[s, k],
                        recv_sem=recv_sems.at[s, k],
                        device_id=(my_x, my_y, my_z),
                        device_id_type=pl.DeviceIdType.MESH,
                    )
                    rx.wait_recv()
            cp = pltpu.make_async_copy(
                recv_buf.at[s],
                out_ref.at[pl.ds(off, PAD)],
                local_sem,
            )
            cp.start()
            cp.wait()

        c_s = jnp.where(
            my_z == s,
            ends_ref[s] - starts_ref[s],
            cnt_smem[s, my_z],
        )
        off = off + c_s

    for item in sends:
        if isinstance(item, tuple):
            rdma, cond = item

            @pl.when(cond)
            def _():
                rdma.wait_send()
        else:
            item.wait_send()
    for rdma in cnt_sends:
        rdma.wait_send()


def kernel(x, dest):
    my_counts = (dest[:, None] == jnp.arange(N_DEV)[None, :]).sum(
        axis=0, dtype=jnp.int32
    )
    zero = jnp.zeros((1,), jnp.int32)
    starts = jnp.concatenate([zero, jnp.cumsum(my_counts)[:-1]]).astype(jnp.int32)
    ends = (starts + my_counts).astype(jnp.int32)
    ls = jnp.argsort(dest, stable=True).astype(jnp.int32)
    cnt_v = my_counts.reshape(1, N_DEV)

    x3 = x.astype(jnp.bfloat16).reshape(T, 8, 128)
    out3 = pl.pallas_call(
        _a2av_body,
        out_shape=jax.ShapeDtypeStruct((T + PAD, 8, 128), jnp.bfloat16),
        in_specs=[
            pl.BlockSpec(memory_space=pltpu.VMEM),
            pl.BlockSpec(memory_space=pltpu.SMEM),
            pl.BlockSpec(memory_space=pltpu.SMEM),
            pl.BlockSpec(memory_space=pltpu.SMEM),
            pl.BlockSpec(memory_space=pltpu.VMEM),
        ],
        out_specs=pl.BlockSpec(memory_space=pltpu.VMEM),
        scratch_shapes=[
            pltpu.VMEM((T + PAD, 8, 128), jnp.bfloat16),
            pltpu.VMEM((N_DEV, PAD, 8, 128), jnp.bfloat16),
            pltpu.VMEM((N_DEV, N_DEV), jnp.int32),
            pltpu.SMEM((N_DEV, N_DEV), jnp.int32),
            pltpu.SemaphoreType.DMA((N_DEV, N_DEV)),
            pltpu.SemaphoreType.DMA((N_DEV, N_DEV)),
            pltpu.SemaphoreType.DMA((N_DEV,)),
            pltpu.SemaphoreType.DMA((N_DEV,)),
            pltpu.SemaphoreType.DMA((N_DEV,)),
            pltpu.SemaphoreType.DMA((N_DEV,)),
            pltpu.SemaphoreType.DMA,
            pltpu.SemaphoreType.DMA,
        ],
        compiler_params=pltpu.CompilerParams(collective_id=0),
    )(x3, ls, starts, ends, cnt_v)

    return out3[:T].reshape(T, D)
